# Initial kernel scaffold; baseline (speedup 1.0000x reference)
import jax
import jax.numpy as jnp
from jax.experimental import pallas as pl


def kernel(x, xyz, W, bn_gamma, bn_beta):
    raise NotImplementedError("write your pallas kernel here")


# speedup vs baseline: 22.1654x; 22.1654x over previous
import functools

import jax
import jax.numpy as jnp
import numpy as np
from jax import lax
from jax.experimental import pallas as pl
from jax.experimental.pallas import tpu as pltpu
from jax.experimental.pallas import tpu_sc as plsc

_RADIUS = 0.2
_R2 = _RADIUS ** 2
_K = 16
_STRIDE = 4
_EPS = 1e-5
_BN_INV = float(1.0 / np.sqrt(np.float32(1.0 + _EPS)))

_NW = 32
_TS = 128


def _fps_body(xs_ref, ys_ref, idx_ref, sx_ref, sy_ref):
    B, N = xs_ref.shape
    S = idx_ref.shape[1]
    xs = xs_ref[...]
    ys = ys_ref[...]
    lane = lax.broadcasted_iota(jnp.int32, (B, N), 1)
    lane_s = lax.broadcasted_iota(jnp.int32, (B, S), 1)

    def step(t, carry):
        dist_acc, far = carry
        sel = lane == far
        cx = jnp.sum(jnp.where(sel, xs, 0.0), axis=1, keepdims=True)
        cy = jnp.sum(jnp.where(sel, ys, 0.0), axis=1, keepdims=True)
        hit = lane_s == t
        idx_ref[...] = jnp.where(hit, far, idx_ref[...])
        sx_ref[...] = jnp.where(hit, cx, sx_ref[...])
        sy_ref[...] = jnp.where(hit, cy, sy_ref[...])
        dx = xs - cx
        dy = ys - cy
        d = dx * dx + dy * dy
        dist_acc = jnp.where(d < dist_acc, d, dist_acc)
        maxv = jnp.max(dist_acc, axis=1, keepdims=True)
        far_new = jnp.min(
            jnp.where(dist_acc == maxv, lane, N), axis=1, keepdims=True
        ).astype(jnp.int32)
        return dist_acc, far_new

    init = (jnp.full((B, N), 1e10, jnp.float32), jnp.zeros((B, 1), jnp.int32))
    lax.fori_loop(0, S, step, init)


def _fps_call(xs, ys, S):
    B, N = xs.shape
    return pl.pallas_call(
        _fps_body,
        out_shape=(
            jax.ShapeDtypeStruct((B, S), jnp.int32),
            jax.ShapeDtypeStruct((B, S), jnp.float32),
            jax.ShapeDtypeStruct((B, S), jnp.float32),
        ),
    )(xs, ys)


def _bq_body(wx0_ref, wx1_ref, xs_ref, ys_ref, sx_ref, sy_ref,
             nidx_ref, fidx_ref, ps_ref):
    b = pl.program_id(0)
    N = xs_ref.shape[2]
    sxv = sx_ref[0]
    syv = sy_ref[0]
    dx = sxv - xs_ref[0]
    dy = syv - ys_ref[0]
    d = dx * dx + dy * dy
    lane = lax.broadcasted_iota(jnp.int32, d.shape, 1)
    key = jnp.where(d <= _R2, lane, N)
    cols = []
    for j in range(_K):
        mn = jnp.min(key, axis=1, keepdims=True)
        cols.append(mn)
        if j < _K - 1:
            key = jnp.where(lane > mn, key, N)
    first = cols[0]
    cols = [first] + [jnp.where(c == N, first, c) for c in cols[1:]]
    nidx = jnp.concatenate(cols, axis=1)
    nidx_ref[0] = nidx
    fidx_ref[0] = nidx + b * N
    ps_ref[0] = (sxv * wx0_ref[...] + syv * wx1_ref[...]) * (1.0 / _RADIUS)


def _bq_call(wx0, wx1, xs3, ys3, sx3, sy3):
    B, _, N = xs3.shape
    S = sx3.shape[1]
    D = wx0.shape[1]
    grid = (B, S // _TS)
    return pl.pallas_call(
        _bq_body,
        grid=grid,
        in_specs=[
            pl.BlockSpec((1, D), lambda b, st: (0, 0)),
            pl.BlockSpec((1, D), lambda b, st: (0, 0)),
            pl.BlockSpec((1, 1, N), lambda b, st: (b, 0, 0)),
            pl.BlockSpec((1, 1, N), lambda b, st: (b, 0, 0)),
            pl.BlockSpec((1, _TS, 1), lambda b, st: (b, st, 0)),
            pl.BlockSpec((1, _TS, 1), lambda b, st: (b, st, 0)),
        ],
        out_specs=[
            pl.BlockSpec((1, _TS, _K), lambda b, st: (b, st, 0)),
            pl.BlockSpec((1, _TS, _K), lambda b, st: (b, st, 0)),
            pl.BlockSpec((1, _TS, D), lambda b, st: (b, st, 0)),
        ],
        out_shape=(
            jax.ShapeDtypeStruct((B, S, _K), jnp.int32),
            jax.ShapeDtypeStruct((B, S, _K), jnp.int32),
            jax.ShapeDtypeStruct((B, S, D), jnp.float32),
        ),
    )(wx0, wx1, xs3, ys3, sx3, sy3)


def _gt_body(wf_ref, wx0_ref, wx1_ref, x_ref, xyzt_ref, g_ref):
    xb = x_ref[0]
    g = lax.dot_general(
        xb, wf_ref[...], (((0,), (1,)), ((), ())),
        preferred_element_type=jnp.float32,
    )
    xsv = xyzt_ref[0, :, 0:1]
    ysv = xyzt_ref[0, :, 1:2]
    g = g + (xsv * wx0_ref[...] + ysv * wx1_ref[...]) * (1.0 / _RADIUS)
    g_ref[0] = jnp.concatenate(
        [g, jnp.zeros_like(g)], axis=1)


def _gt_call(wf, wx0, wx1, x, xyzt):
    B, D, N = x.shape
    return pl.pallas_call(
        _gt_body,
        grid=(B,),
        in_specs=[
            pl.BlockSpec((D, D), lambda b: (0, 0)),
            pl.BlockSpec((1, D), lambda b: (0, 0)),
            pl.BlockSpec((1, D), lambda b: (0, 0)),
            pl.BlockSpec((1, D, N), lambda b: (b, 0, 0)),
            pl.BlockSpec((1, N, 2), lambda b: (b, 0, 0)),
        ],
        out_specs=pl.BlockSpec((1, N, 2 * D), lambda b: (b, 0, 0)),
        out_shape=jax.ShapeDtypeStruct((B, N, 2 * D), jnp.float32),
    )(wf, wx0, wx1, x, xyzt)


def _sc_gather_max(table, fidx, ps, gamma, beta, B, S, D):
    SPW = (B * S) // _NW
    CH = 8
    NCH = SPW // CH
    ROWS = CH * _K
    mesh = plsc.VectorSubcoreMesh(core_axis_name="c", subcore_axis_name="s")

    @functools.partial(
        pl.kernel,
        out_type=jax.ShapeDtypeStruct((B * S, D), jnp.float32),
        mesh=mesh,
        scratch_types=[
            pltpu.VMEM((SPW * _K,), jnp.int32),
            pltpu.VMEM((ROWS, 2 * D), jnp.float32),
            pltpu.VMEM((ROWS, 2 * D), jnp.float32),
            pltpu.VMEM((SPW, D), jnp.float32),
            pltpu.VMEM((D,), jnp.float32),
            pltpu.VMEM((D,), jnp.float32),
            pltpu.VMEM((SPW, D), jnp.float32),
            pltpu.SemaphoreType.DMA,
            pltpu.SemaphoreType.DMA,
        ],
    )
    def k(table_h, fidx_h, ps_h, gamma_h, beta_h, out_h,
          idx_v, rows0, rows1, psb, gmv, btv, resb, sem0, sem1):
        cid = lax.axis_index("c")
        sid = lax.axis_index("s")
        wid = sid * 2 + cid
        base = wid * SPW
        pltpu.sync_copy(fidx_h.at[pl.ds(base * _K, SPW * _K)], idx_v)
        pltpu.sync_copy(ps_h.at[pl.ds(base, SPW)], psb)
        pltpu.sync_copy(gamma_h, gmv)
        pltpu.sync_copy(beta_h, btv)

        def start(c, buf, sem):
            pltpu.async_copy(
                table_h.at[idx_v.at[pl.ds(c * ROWS, ROWS)]], buf, sem)

        def wait(c, buf, sem):
            pltpu.make_async_copy(
                table_h.at[idx_v.at[pl.ds(c * ROWS, ROWS)]], buf, sem).wait()

        def compute(c, buf):
            for i in range(CH):
                s_local = c * CH + i
                for f in range(D // 16):
                    fo = f * 16
                    acc = buf[i * _K, pl.ds(fo, 16)]
                    for r in range(1, _K):
                        acc = jnp.maximum(acc, buf[i * _K + r, pl.ds(fo, 16)])
                    acc = acc - psb[s_local, pl.ds(fo, 16)]
                    acc = acc * (gmv[pl.ds(fo, 16)] * _BN_INV) + btv[pl.ds(fo, 16)]
                    acc = jnp.maximum(acc, 0.0)
                    resb[s_local, pl.ds(fo, 16)] = acc

        start(0, rows0, sem0)
        start(1, rows1, sem1)

        def loop_body(g, carry):
            c0 = g * 2
            wait(c0, rows0, sem0)
            compute(c0, rows0)

            @pl.when(g < NCH // 2 - 1)
            def _():
                start(c0 + 2, rows0, sem0)

            wait(c0 + 1, rows1, sem1)
            compute(c0 + 1, rows1)

            @pl.when(g < NCH // 2 - 1)
            def _():
                start(c0 + 3, rows1, sem1)

            return carry

        lax.fori_loop(0, NCH // 2, loop_body, 0)
        pltpu.sync_copy(resb, out_h.at[pl.ds(base, SPW)])

    return k(table, fidx, ps, gamma, beta)


def _tr_body(m_ref, out_ref):
    out_ref[0] = jnp.transpose(m_ref[0], (1, 0))


def _tr_call(m, B, S, D):
    return pl.pallas_call(
        _tr_body,
        grid=(B,),
        in_specs=[pl.BlockSpec((1, S, D), lambda b: (b, 0, 0))],
        out_specs=pl.BlockSpec((1, D, S), lambda b: (b, 0, 0)),
        out_shape=jax.ShapeDtypeStruct((B, D, S), jnp.float32),
    )(m)


def kernel(x, xyz, W, bn_gamma, bn_beta):
    B, D, N = x.shape
    S = N // _STRIDE
    xs = xyz[:, 0, :]
    ys = xyz[:, 1, :]
    sample_idx, sx, sy = _fps_call(xs, ys, S)
    sample_xyz = jnp.stack([sx, sy], axis=1)
    wf = W[:, :D]
    wx0 = W[:, D][None, :]
    wx1 = W[:, D + 1][None, :]
    nidx, fidx, ps = _bq_call(wx0, wx1, xs[:, None, :], ys[:, None, :],
                              sx[:, :, None], sy[:, :, None])
    g = _gt_call(wf, wx0, wx1, x, jnp.transpose(xyz, (0, 2, 1)))
    m = _sc_gather_max(
        g.reshape(B * N, 2 * D),
        fidx.reshape(B * S * _K),
        ps.reshape(B * S, D),
        bn_gamma, bn_beta, B, S, D,
    )
    out = _tr_call(m.reshape(B, S, D), B, S, D)
    return out, sample_xyz, sample_idx, nidx


# speedup vs baseline: 24.8710x; 1.1221x over previous
import functools

import jax
import jax.numpy as jnp
import numpy as np
from jax import lax
from jax.experimental import pallas as pl
from jax.experimental.pallas import tpu as pltpu
from jax.experimental.pallas import tpu_sc as plsc

_RADIUS = 0.2
_R2 = _RADIUS ** 2
_K = 16
_STRIDE = 4
_EPS = 1e-5
_BN_INV = float(1.0 / np.sqrt(np.float32(1.0 + _EPS)))

_NW = 32
_TS = 128


def _fps_body(xs_ref, ys_ref, idx_ref, sx_ref, sy_ref):
    B, N = xs_ref.shape
    S = idx_ref.shape[1]
    xs = xs_ref[...]
    ys = ys_ref[...]
    lane = lax.broadcasted_iota(jnp.int32, (B, N), 1)
    lane_s = lax.broadcasted_iota(jnp.int32, (B, S), 1)

    def step(t, carry):
        dist_acc, far = carry
        sel = lane == far
        cx = jnp.sum(jnp.where(sel, xs, 0.0), axis=1, keepdims=True)
        cy = jnp.sum(jnp.where(sel, ys, 0.0), axis=1, keepdims=True)
        hit = lane_s == t
        idx_ref[...] = jnp.where(hit, far, idx_ref[...])
        sx_ref[...] = jnp.where(hit, cx, sx_ref[...])
        sy_ref[...] = jnp.where(hit, cy, sy_ref[...])
        dx = xs - cx
        dy = ys - cy
        d = dx * dx + dy * dy
        dist_acc = jnp.where(d < dist_acc, d, dist_acc)
        maxv = jnp.max(dist_acc, axis=1, keepdims=True)
        far_new = jnp.min(
            jnp.where(dist_acc == maxv, lane, N), axis=1, keepdims=True
        ).astype(jnp.int32)
        return dist_acc, far_new

    init = (jnp.full((B, N), 1e10, jnp.float32), jnp.zeros((B, 1), jnp.int32))
    lax.fori_loop(0, S, step, init)


def _fps_call(xs, ys, S):
    B, N = xs.shape
    return pl.pallas_call(
        _fps_body,
        out_shape=(
            jax.ShapeDtypeStruct((B, S), jnp.int32),
            jax.ShapeDtypeStruct((B, S), jnp.float32),
            jax.ShapeDtypeStruct((B, S), jnp.float32),
        ),
    )(xs, ys)


_PB = 16


def _bq_body(wx0_ref, wx1_ref, pmat_ref, xs_ref, ys_ref, sx_ref, sy_ref,
             nidx_ref, fidx_ref, ps_ref):
    b = pl.program_id(0)
    N = xs_ref.shape[2]
    NL = N // _PB
    sxv = sx_ref[0]
    syv = sy_ref[0]
    dx = sxv - xs_ref[0]
    dy = syv - ys_ref[0]
    d = dx * dx + dy * dy
    mask = jnp.where(d <= _R2, 1.0, 0.0)
    pk = lax.dot_general(
        mask, pmat_ref[...], (((1,), (0,)), ((), ())),
        preferred_element_type=jnp.float32,
    )
    p = pk.astype(jnp.int32)
    lane = lax.broadcasted_iota(jnp.int32, p.shape, 1)
    cols = []
    first = None
    for j in range(_K):
        key = jnp.where(p > 0, lane, NL)
        mn = jnp.min(key, axis=1, keepdims=True)
        oneh = key == mn
        v = jnp.sum(jnp.where(oneh, p, 0), axis=1, keepdims=True)
        lsb = jnp.bitwise_and(v, -v)
        bpos = lax.shift_right_logical(
            lax.bitcast_convert_type(lsb.astype(jnp.float32), jnp.int32), 23
        ) - 127
        idx = mn * _PB + bpos
        if first is None:
            first = idx
            cols.append(idx)
        else:
            cols.append(jnp.where(mn < NL, idx, first))
        if j < _K - 1:
            p = jnp.where(oneh, v - lsb, p)
    nidx = jnp.concatenate(cols, axis=1)
    nidx_ref[0] = nidx
    fidx_ref[0] = nidx + b * N
    ps_ref[0] = (sxv * wx0_ref[...] + syv * wx1_ref[...]) * (1.0 / _RADIUS)


def _bq_call(wx0, wx1, pmat, xs3, ys3, sx3, sy3):
    B, _, N = xs3.shape
    S = sx3.shape[1]
    D = wx0.shape[1]
    NL = N // _PB
    grid = (B, S // _TS)
    return pl.pallas_call(
        _bq_body,
        grid=grid,
        in_specs=[
            pl.BlockSpec((1, D), lambda b, st: (0, 0)),
            pl.BlockSpec((1, D), lambda b, st: (0, 0)),
            pl.BlockSpec((N, NL), lambda b, st: (0, 0)),
            pl.BlockSpec((1, 1, N), lambda b, st: (b, 0, 0)),
            pl.BlockSpec((1, 1, N), lambda b, st: (b, 0, 0)),
            pl.BlockSpec((1, _TS, 1), lambda b, st: (b, st, 0)),
            pl.BlockSpec((1, _TS, 1), lambda b, st: (b, st, 0)),
        ],
        out_specs=[
            pl.BlockSpec((1, _TS, _K), lambda b, st: (b, st, 0)),
            pl.BlockSpec((1, _TS, _K), lambda b, st: (b, st, 0)),
            pl.BlockSpec((1, _TS, D), lambda b, st: (b, st, 0)),
        ],
        out_shape=(
            jax.ShapeDtypeStruct((B, S, _K), jnp.int32),
            jax.ShapeDtypeStruct((B, S, _K), jnp.int32),
            jax.ShapeDtypeStruct((B, S, D), jnp.float32),
        ),
    )(wx0, wx1, pmat, xs3, ys3, sx3, sy3)


def _gt_body(wf_ref, wx0_ref, wx1_ref, x_ref, xyzt_ref, g_ref):
    xb = x_ref[0]
    g = lax.dot_general(
        xb, wf_ref[...], (((0,), (1,)), ((), ())),
        preferred_element_type=jnp.float32,
    )
    xsv = xyzt_ref[0, :, 0:1]
    ysv = xyzt_ref[0, :, 1:2]
    g = g + (xsv * wx0_ref[...] + ysv * wx1_ref[...]) * (1.0 / _RADIUS)
    g_ref[0] = jnp.concatenate(
        [g, jnp.zeros_like(g)], axis=1)


def _gt_call(wf, wx0, wx1, x, xyzt):
    B, D, N = x.shape
    return pl.pallas_call(
        _gt_body,
        grid=(B,),
        in_specs=[
            pl.BlockSpec((D, D), lambda b: (0, 0)),
            pl.BlockSpec((1, D), lambda b: (0, 0)),
            pl.BlockSpec((1, D), lambda b: (0, 0)),
            pl.BlockSpec((1, D, N), lambda b: (b, 0, 0)),
            pl.BlockSpec((1, N, 2), lambda b: (b, 0, 0)),
        ],
        out_specs=pl.BlockSpec((1, N, 2 * D), lambda b: (b, 0, 0)),
        out_shape=jax.ShapeDtypeStruct((B, N, 2 * D), jnp.float32),
    )(wf, wx0, wx1, x, xyzt)


def _sc_gather_max(table, fidx, ps, gamma, beta, B, S, D):
    SPW = (B * S) // _NW
    CH = 8
    NCH = SPW // CH
    ROWS = CH * _K
    mesh = plsc.VectorSubcoreMesh(core_axis_name="c", subcore_axis_name="s")

    @functools.partial(
        pl.kernel,
        out_type=jax.ShapeDtypeStruct((B * S, D), jnp.float32),
        mesh=mesh,
        scratch_types=[
            pltpu.VMEM((SPW * _K,), jnp.int32),
            pltpu.VMEM((ROWS, 2 * D), jnp.float32),
            pltpu.VMEM((ROWS, 2 * D), jnp.float32),
            pltpu.VMEM((SPW, D), jnp.float32),
            pltpu.VMEM((D,), jnp.float32),
            pltpu.VMEM((D,), jnp.float32),
            pltpu.VMEM((SPW, D), jnp.float32),
            pltpu.SemaphoreType.DMA,
            pltpu.SemaphoreType.DMA,
        ],
    )
    def k(table_h, fidx_h, ps_h, gamma_h, beta_h, out_h,
          idx_v, rows0, rows1, psb, gmv, btv, resb, sem0, sem1):
        cid = lax.axis_index("c")
        sid = lax.axis_index("s")
        wid = sid * 2 + cid
        base = wid * SPW
        pltpu.sync_copy(fidx_h.at[pl.ds(base * _K, SPW * _K)], idx_v)
        pltpu.sync_copy(ps_h.at[pl.ds(base, SPW)], psb)
        pltpu.sync_copy(gamma_h, gmv)
        pltpu.sync_copy(beta_h, btv)

        def start(c, buf, sem):
            pltpu.async_copy(
                table_h.at[idx_v.at[pl.ds(c * ROWS, ROWS)]], buf, sem)

        def wait(c, buf, sem):
            pltpu.make_async_copy(
                table_h.at[idx_v.at[pl.ds(c * ROWS, ROWS)]], buf, sem).wait()

        def compute(c, buf):
            for i in range(CH):
                s_local = c * CH + i
                for f in range(D // 16):
                    fo = f * 16
                    acc = buf[i * _K, pl.ds(fo, 16)]
                    for r in range(1, _K):
                        acc = jnp.maximum(acc, buf[i * _K + r, pl.ds(fo, 16)])
                    acc = acc - psb[s_local, pl.ds(fo, 16)]
                    acc = acc * (gmv[pl.ds(fo, 16)] * _BN_INV) + btv[pl.ds(fo, 16)]
                    acc = jnp.maximum(acc, 0.0)
                    resb[s_local, pl.ds(fo, 16)] = acc

        start(0, rows0, sem0)
        start(1, rows1, sem1)

        def loop_body(g, carry):
            c0 = g * 2
            wait(c0, rows0, sem0)
            compute(c0, rows0)

            @pl.when(g < NCH // 2 - 1)
            def _():
                start(c0 + 2, rows0, sem0)

            wait(c0 + 1, rows1, sem1)
            compute(c0 + 1, rows1)

            @pl.when(g < NCH // 2 - 1)
            def _():
                start(c0 + 3, rows1, sem1)

            return carry

        lax.fori_loop(0, NCH // 2, loop_body, 0)
        pltpu.sync_copy(resb, out_h.at[pl.ds(base, SPW)])

    return k(table, fidx, ps, gamma, beta)


def _tr_body(m_ref, out_ref):
    out_ref[0] = jnp.transpose(m_ref[0], (1, 0))


def _tr_call(m, B, S, D):
    return pl.pallas_call(
        _tr_body,
        grid=(B,),
        in_specs=[pl.BlockSpec((1, S, D), lambda b: (b, 0, 0))],
        out_specs=pl.BlockSpec((1, D, S), lambda b: (b, 0, 0)),
        out_shape=jax.ShapeDtypeStruct((B, D, S), jnp.float32),
    )(m)


def kernel(x, xyz, W, bn_gamma, bn_beta):
    B, D, N = x.shape
    S = N // _STRIDE
    xs = xyz[:, 0, :]
    ys = xyz[:, 1, :]
    sample_idx, sx, sy = _fps_call(xs, ys, S)
    sample_xyz = jnp.stack([sx, sy], axis=1)
    wf = W[:, :D]
    wx0 = W[:, D][None, :]
    wx1 = W[:, D + 1][None, :]
    n_ar = np.arange(N)
    pmat_np = np.zeros((N, N // _PB), np.float32)
    pmat_np[n_ar, n_ar // _PB] = np.float32(2.0) ** (n_ar % _PB)
    pmat = jnp.asarray(pmat_np)
    nidx, fidx, ps = _bq_call(wx0, wx1, pmat, xs[:, None, :], ys[:, None, :],
                              sx[:, :, None], sy[:, :, None])
    g = _gt_call(wf, wx0, wx1, x, jnp.transpose(xyz, (0, 2, 1)))
    m = _sc_gather_max(
        g.reshape(B * N, 2 * D),
        fidx.reshape(B * S * _K),
        ps.reshape(B * S, D),
        bn_gamma, bn_beta, B, S, D,
    )
    out = _tr_call(m.reshape(B, S, D), B, S, D)
    return out, sample_xyz, sample_idx, nidx


# speedup vs baseline: 30.1696x; 1.2130x over previous
import functools

import jax
import jax.numpy as jnp
import numpy as np
from jax import lax
from jax.experimental import pallas as pl
from jax.experimental.pallas import tpu as pltpu
from jax.experimental.pallas import tpu_sc as plsc

_RADIUS = 0.2
_R2 = _RADIUS ** 2
_K = 16
_STRIDE = 4
_EPS = 1e-5
_BN_INV = float(1.0 / np.sqrt(np.float32(1.0 + _EPS)))

_NW = 32
_TS = 128


def _fps_body(xs_ref, ys_ref, idx_ref, sx_ref, sy_ref):
    B, N = xs_ref.shape
    S = idx_ref.shape[1]
    xs = xs_ref[...]
    ys = ys_ref[...]
    lane = lax.broadcasted_iota(jnp.int32, (B, N), 1)
    lane_s = lax.broadcasted_iota(jnp.int32, (B, S), 1)

    def step(t, carry):
        dist_acc, far = carry
        sel = lane == far
        cx = jnp.sum(jnp.where(sel, xs, 0.0), axis=1, keepdims=True)
        cy = jnp.sum(jnp.where(sel, ys, 0.0), axis=1, keepdims=True)
        hit = lane_s == t
        idx_ref[...] = jnp.where(hit, far, idx_ref[...])
        sx_ref[...] = jnp.where(hit, cx, sx_ref[...])
        sy_ref[...] = jnp.where(hit, cy, sy_ref[...])
        dx = xs - cx
        dy = ys - cy
        d = dx * dx + dy * dy
        dist_acc = jnp.where(d < dist_acc, d, dist_acc)
        maxv = jnp.max(dist_acc, axis=1, keepdims=True)
        far_new = jnp.min(
            jnp.where(dist_acc == maxv, lane, N), axis=1, keepdims=True
        ).astype(jnp.int32)
        return dist_acc, far_new

    init = (jnp.full((B, N), 1e10, jnp.float32), jnp.zeros((B, 1), jnp.int32))
    lax.fori_loop(0, S, step, init)


def _fps_call(xs, ys, S):
    B, N = xs.shape
    return pl.pallas_call(
        _fps_body,
        out_shape=(
            jax.ShapeDtypeStruct((B, S), jnp.int32),
            jax.ShapeDtypeStruct((B, S), jnp.float32),
            jax.ShapeDtypeStruct((B, S), jnp.float32),
        ),
    )(xs, ys)


_PB = 16


def _bq_body(wx0_ref, wx1_ref, pmat_ref, xc_ref, yc_ref, sxr_ref, syr_ref,
             sxc_ref, syc_ref, nidx_ref, fidx_ref, ps_ref):
    b = pl.program_id(0)
    N = xc_ref.shape[1]
    NL = N // _PB
    xc = xc_ref[0]
    yc = yc_ref[0]
    sxr = sxr_ref[0]
    syr = syr_ref[0]
    dx = xc - sxr
    dy = yc - syr
    d = dx * dx + dy * dy
    mask = jnp.where(d <= _R2, 1.0, 0.0)
    pk = lax.dot_general(
        pmat_ref[...], mask, (((1,), (0,)), ((), ())),
        preferred_element_type=jnp.float32,
    )
    p = pk.astype(jnp.int32)
    subl = lax.broadcasted_iota(jnp.int32, p.shape, 0)
    big = jnp.int32(0x7FFFFFFF)
    comb = jnp.where(p > 0, (subl << 16) + p, big)
    cols = []
    first = None
    for j in range(_K):
        mc = jnp.min(comb, axis=0, keepdims=True)
        v = jnp.bitwise_and(mc, 0xFFFF)
        mnl = lax.shift_right_logical(mc, 16)
        lsb = jnp.bitwise_and(v, -v)
        bpos = lax.shift_right_logical(
            lax.bitcast_convert_type(lsb.astype(jnp.float32), jnp.int32), 23
        ) - 127
        idx = mnl * _PB + bpos
        if first is None:
            first = idx
            cols.append(idx)
        else:
            cols.append(jnp.where(mnl < NL, idx, first))
        if j < _K - 1:
            oneh = subl == mnl
            newc = jnp.where(v == lsb, big, mc - lsb)
            comb = jnp.where(oneh, newc, comb)
    nidx = jnp.transpose(jnp.concatenate(cols, axis=0), (1, 0))
    nidx_ref[0] = nidx
    fidx_ref[0] = nidx + b * N
    sxv = sxc_ref[0]
    syv = syc_ref[0]
    ps_ref[0] = (sxv * wx0_ref[...] + syv * wx1_ref[...]) * (1.0 / _RADIUS)


def _bq_call(wx0, wx1, pmat, xcol, ycol, sxr, syr, sxc, syc):
    B, N, _ = xcol.shape
    S = sxr.shape[2]
    D = wx0.shape[1]
    NL = N // _PB
    grid = (B, S // _TS)
    return pl.pallas_call(
        _bq_body,
        grid=grid,
        in_specs=[
            pl.BlockSpec((1, D), lambda b, st: (0, 0)),
            pl.BlockSpec((1, D), lambda b, st: (0, 0)),
            pl.BlockSpec((NL, N), lambda b, st: (0, 0)),
            pl.BlockSpec((1, N, 1), lambda b, st: (b, 0, 0)),
            pl.BlockSpec((1, N, 1), lambda b, st: (b, 0, 0)),
            pl.BlockSpec((1, 1, _TS), lambda b, st: (b, 0, st)),
            pl.BlockSpec((1, 1, _TS), lambda b, st: (b, 0, st)),
            pl.BlockSpec((1, _TS, 1), lambda b, st: (b, st, 0)),
            pl.BlockSpec((1, _TS, 1), lambda b, st: (b, st, 0)),
        ],
        out_specs=[
            pl.BlockSpec((1, _TS, _K), lambda b, st: (b, st, 0)),
            pl.BlockSpec((1, _TS, _K), lambda b, st: (b, st, 0)),
            pl.BlockSpec((1, _TS, D), lambda b, st: (b, st, 0)),
        ],
        out_shape=(
            jax.ShapeDtypeStruct((B, S, _K), jnp.int32),
            jax.ShapeDtypeStruct((B, S, _K), jnp.int32),
            jax.ShapeDtypeStruct((B, S, D), jnp.float32),
        ),
    )(wx0, wx1, pmat, xcol, ycol, sxr, syr, sxc, syc)


def _gt_body(wf_ref, wx0_ref, wx1_ref, x_ref, xyzt_ref, g_ref):
    xb = x_ref[0]
    g = lax.dot_general(
        xb, wf_ref[...], (((0,), (1,)), ((), ())),
        preferred_element_type=jnp.float32,
    )
    xsv = xyzt_ref[0, :, 0:1]
    ysv = xyzt_ref[0, :, 1:2]
    g = g + (xsv * wx0_ref[...] + ysv * wx1_ref[...]) * (1.0 / _RADIUS)
    g_ref[0] = jnp.concatenate(
        [g, jnp.zeros_like(g)], axis=1)


def _gt_call(wf, wx0, wx1, x, xyzt):
    B, D, N = x.shape
    return pl.pallas_call(
        _gt_body,
        grid=(B,),
        in_specs=[
            pl.BlockSpec((D, D), lambda b: (0, 0)),
            pl.BlockSpec((1, D), lambda b: (0, 0)),
            pl.BlockSpec((1, D), lambda b: (0, 0)),
            pl.BlockSpec((1, D, N), lambda b: (b, 0, 0)),
            pl.BlockSpec((1, N, 2), lambda b: (b, 0, 0)),
        ],
        out_specs=pl.BlockSpec((1, N, 2 * D), lambda b: (b, 0, 0)),
        out_shape=jax.ShapeDtypeStruct((B, N, 2 * D), jnp.float32),
    )(wf, wx0, wx1, x, xyzt)


def _sc_gather_max(table, fidx, ps, gamma, beta, B, S, D):
    SPW = (B * S) // _NW
    CH = 8
    NCH = SPW // CH
    ROWS = CH * _K
    mesh = plsc.VectorSubcoreMesh(core_axis_name="c", subcore_axis_name="s")

    @functools.partial(
        pl.kernel,
        out_type=jax.ShapeDtypeStruct((B * S, D), jnp.float32),
        mesh=mesh,
        scratch_types=[
            pltpu.VMEM((SPW * _K,), jnp.int32),
            pltpu.VMEM((ROWS, 2 * D), jnp.float32),
            pltpu.VMEM((ROWS, 2 * D), jnp.float32),
            pltpu.VMEM((SPW, D), jnp.float32),
            pltpu.VMEM((D,), jnp.float32),
            pltpu.VMEM((D,), jnp.float32),
            pltpu.VMEM((SPW, D), jnp.float32),
            pltpu.SemaphoreType.DMA,
            pltpu.SemaphoreType.DMA,
        ],
    )
    def k(table_h, fidx_h, ps_h, gamma_h, beta_h, out_h,
          idx_v, rows0, rows1, psb, gmv, btv, resb, sem0, sem1):
        cid = lax.axis_index("c")
        sid = lax.axis_index("s")
        wid = sid * 2 + cid
        base = wid * SPW
        pltpu.sync_copy(fidx_h.at[pl.ds(base * _K, SPW * _K)], idx_v)
        pltpu.sync_copy(ps_h.at[pl.ds(base, SPW)], psb)
        pltpu.sync_copy(gamma_h, gmv)
        pltpu.sync_copy(beta_h, btv)

        def start(c, buf, sem):
            pltpu.async_copy(
                table_h.at[idx_v.at[pl.ds(c * ROWS, ROWS)]], buf, sem)

        def wait(c, buf, sem):
            pltpu.make_async_copy(
                table_h.at[idx_v.at[pl.ds(c * ROWS, ROWS)]], buf, sem).wait()

        def compute(c, buf):
            for i in range(CH):
                s_local = c * CH + i
                for f in range(D // 16):
                    fo = f * 16
                    acc = buf[i * _K, pl.ds(fo, 16)]
                    for r in range(1, _K):
                        acc = jnp.maximum(acc, buf[i * _K + r, pl.ds(fo, 16)])
                    acc = acc - psb[s_local, pl.ds(fo, 16)]
                    acc = acc * (gmv[pl.ds(fo, 16)] * _BN_INV) + btv[pl.ds(fo, 16)]
                    acc = jnp.maximum(acc, 0.0)
                    resb[s_local, pl.ds(fo, 16)] = acc

        start(0, rows0, sem0)
        start(1, rows1, sem1)

        def loop_body(g, carry):
            c0 = g * 2
            wait(c0, rows0, sem0)
            compute(c0, rows0)

            @pl.when(g < NCH // 2 - 1)
            def _():
                start(c0 + 2, rows0, sem0)

            wait(c0 + 1, rows1, sem1)
            compute(c0 + 1, rows1)

            @pl.when(g < NCH // 2 - 1)
            def _():
                start(c0 + 3, rows1, sem1)

            return carry

        lax.fori_loop(0, NCH // 2, loop_body, 0)
        pltpu.sync_copy(resb, out_h.at[pl.ds(base, SPW)])

    return k(table, fidx, ps, gamma, beta)


def _tr_body(m_ref, out_ref):
    out_ref[0] = jnp.transpose(m_ref[0], (1, 0))


def _tr_call(m, B, S, D):
    return pl.pallas_call(
        _tr_body,
        grid=(B,),
        in_specs=[pl.BlockSpec((1, S, D), lambda b: (b, 0, 0))],
        out_specs=pl.BlockSpec((1, D, S), lambda b: (b, 0, 0)),
        out_shape=jax.ShapeDtypeStruct((B, D, S), jnp.float32),
    )(m)


def kernel(x, xyz, W, bn_gamma, bn_beta):
    B, D, N = x.shape
    S = N // _STRIDE
    xs = xyz[:, 0, :]
    ys = xyz[:, 1, :]
    sample_idx, sx, sy = _fps_call(xs, ys, S)
    sample_xyz = jnp.stack([sx, sy], axis=1)
    wf = W[:, :D]
    wx0 = W[:, D][None, :]
    wx1 = W[:, D + 1][None, :]
    n_ar = np.arange(N)
    pmat_np = np.zeros((N // _PB, N), np.float32)
    pmat_np[n_ar // _PB, n_ar] = np.float32(2.0) ** (n_ar % _PB)
    pmat = jnp.asarray(pmat_np)
    nidx, fidx, ps = _bq_call(wx0, wx1, pmat,
                              xs[:, :, None], ys[:, :, None],
                              sx[:, None, :], sy[:, None, :],
                              sx[:, :, None], sy[:, :, None])
    g = _gt_call(wf, wx0, wx1, x, jnp.transpose(xyz, (0, 2, 1)))
    m = _sc_gather_max(
        g.reshape(B * N, 2 * D),
        fidx.reshape(B * S * _K),
        ps.reshape(B * S, D),
        bn_gamma, bn_beta, B, S, D,
    )
    out = _tr_call(m.reshape(B, S, D), B, S, D)
    return out, sample_xyz, sample_idx, nidx


# speedup vs baseline: 32.3359x; 1.0718x over previous
import functools

import jax
import jax.numpy as jnp
import numpy as np
from jax import lax
from jax.experimental import pallas as pl
from jax.experimental.pallas import tpu as pltpu
from jax.experimental.pallas import tpu_sc as plsc

_RADIUS = 0.2
_R2 = _RADIUS ** 2
_K = 16
_STRIDE = 4
_EPS = 1e-5
_BN_INV = float(1.0 / np.sqrt(np.float32(1.0 + _EPS)))

_NW = 32
_TS = 128


_FW = 1024
_FB = 128


def _fps_body(xs_ref, ys_ref, idx_ref, sx_ref, sy_ref, dist_ref):
    B, N = xs_ref.shape
    S = idx_ref.shape[1]
    NC = N // _FW
    for c in range(NC):
        dist_ref[:, pl.ds(c * _FW, _FW)] = jnp.full((B, _FW), 1e10,
                                                    jnp.float32)
    lane_b = lax.broadcasted_iota(jnp.int32, (B, _FB), 1)

    def step(t, carry):
        far, cx, cy, bufi, bufx, bufy = carry
        tm = jnp.bitwise_and(t, _FB - 1)
        hit = lane_b == tm
        bufi = jnp.where(hit, far, bufi)
        bufx = jnp.where(hit, cx, bufx)
        bufy = jnp.where(hit, cy, bufy)

        ms, idxs, cxs, cys = [], [], [], []
        for c in range(NC):
            sl = pl.ds(c * _FW, _FW)
            xc = xs_ref[:, sl]
            yc = ys_ref[:, sl]
            dc = dist_ref[:, sl]
            dxc = xc - cx
            dyc = yc - cy
            dd = dxc * dxc + dyc * dyc
            dc = jnp.where(dd < dc, dd, dc)
            dist_ref[:, sl] = dc
            lane_c = lax.broadcasted_iota(jnp.int32, (B, _FW), 1) + c * _FW
            mc = jnp.max(dc, axis=1, keepdims=True)
            ic = jnp.min(jnp.where(dc == mc, lane_c, N), axis=1,
                         keepdims=True)
            oh = lane_c == ic
            ms.append(mc)
            idxs.append(ic)
            cxs.append(jnp.sum(jnp.where(oh, xc, 0.0), axis=1, keepdims=True))
            cys.append(jnp.sum(jnp.where(oh, yc, 0.0), axis=1, keepdims=True))
        gmax = ms[0]
        for c in range(1, NC):
            gmax = jnp.maximum(gmax, ms[c])
        gidx = None
        for c in range(NC):
            cand = jnp.where(ms[c] == gmax, idxs[c], N)
            gidx = cand if gidx is None else jnp.minimum(gidx, cand)
        ncx = jnp.zeros((B, 1), jnp.float32)
        ncy = jnp.zeros((B, 1), jnp.float32)
        for c in range(NC):
            selc = (ms[c] == gmax) & (idxs[c] == gidx)
            ncx = ncx + jnp.where(selc, cxs[c], 0.0)
            ncy = ncy + jnp.where(selc, cys[c], 0.0)

        blk = pl.multiple_of(jnp.bitwise_and(t, ~(_FB - 1)), _FB)
        idx_ref[:, pl.ds(blk, _FB)] = bufi
        sx_ref[:, pl.ds(blk, _FB)] = bufx
        sy_ref[:, pl.ds(blk, _FB)] = bufy
        return gidx, ncx, ncy, bufi, bufx, bufy

    first_m = lane_b == 0
    init = (
        jnp.min(jnp.where(first_m, lane_b, N), axis=1, keepdims=True),
        jnp.sum(jnp.where(first_m, xs_ref[:, 0:_FB], 0.0), axis=1,
                keepdims=True),
        jnp.sum(jnp.where(first_m, ys_ref[:, 0:_FB], 0.0), axis=1,
                keepdims=True),
        xs_ref[:, 0:_FB].astype(jnp.int32),
        xs_ref[:, 0:_FB],
        ys_ref[:, 0:_FB],
    )
    lax.fori_loop(0, S, step, init)


def _fps_call(xs, ys, S):
    B, N = xs.shape
    return pl.pallas_call(
        _fps_body,
        out_shape=(
            jax.ShapeDtypeStruct((B, S), jnp.int32),
            jax.ShapeDtypeStruct((B, S), jnp.float32),
            jax.ShapeDtypeStruct((B, S), jnp.float32),
        ),
        scratch_shapes=[pltpu.VMEM((B, N), jnp.float32)],
    )(xs, ys)


_PB = 16


def _bq_body(wx0_ref, wx1_ref, pmat_ref, xc_ref, yc_ref, sxr_ref, syr_ref,
             sxc_ref, syc_ref, nidx_ref, fidx_ref, ps_ref):
    b = pl.program_id(0)
    N = xc_ref.shape[1]
    NL = N // _PB
    xc = xc_ref[0]
    yc = yc_ref[0]
    sxr = sxr_ref[0]
    syr = syr_ref[0]
    dx = xc - sxr
    dy = yc - syr
    d = dx * dx + dy * dy
    mask = jnp.where(d <= _R2, 1.0, 0.0)
    pk = lax.dot_general(
        pmat_ref[...], mask, (((1,), (0,)), ((), ())),
        preferred_element_type=jnp.float32,
    )
    p = pk.astype(jnp.int32)
    subl = lax.broadcasted_iota(jnp.int32, p.shape, 0)
    big = jnp.int32(0x7FFFFFFF)
    comb = jnp.where(p > 0, (subl << 16) + p, big)
    cols = []
    first = None
    for j in range(_K):
        mc = jnp.min(comb, axis=0, keepdims=True)
        v = jnp.bitwise_and(mc, 0xFFFF)
        mnl = lax.shift_right_logical(mc, 16)
        lsb = jnp.bitwise_and(v, -v)
        bpos = lax.shift_right_logical(
            lax.bitcast_convert_type(lsb.astype(jnp.float32), jnp.int32), 23
        ) - 127
        idx = mnl * _PB + bpos
        if first is None:
            first = idx
            cols.append(idx)
        else:
            cols.append(jnp.where(mnl < NL, idx, first))
        if j < _K - 1:
            oneh = subl == mnl
            newc = jnp.where(v == lsb, big, mc - lsb)
            comb = jnp.where(oneh, newc, comb)
    nidx = jnp.transpose(jnp.concatenate(cols, axis=0), (1, 0))
    nidx_ref[0] = nidx
    fidx_ref[0] = nidx + b * N
    sxv = sxc_ref[0]
    syv = syc_ref[0]
    ps_ref[0] = (sxv * wx0_ref[...] + syv * wx1_ref[...]) * (1.0 / _RADIUS)


def _bq_call(wx0, wx1, pmat, xcol, ycol, sxr, syr, sxc, syc):
    B, N, _ = xcol.shape
    S = sxr.shape[2]
    D = wx0.shape[1]
    NL = N // _PB
    grid = (B, S // _TS)
    return pl.pallas_call(
        _bq_body,
        grid=grid,
        in_specs=[
            pl.BlockSpec((1, D), lambda b, st: (0, 0)),
            pl.BlockSpec((1, D), lambda b, st: (0, 0)),
            pl.BlockSpec((NL, N), lambda b, st: (0, 0)),
            pl.BlockSpec((1, N, 1), lambda b, st: (b, 0, 0)),
            pl.BlockSpec((1, N, 1), lambda b, st: (b, 0, 0)),
            pl.BlockSpec((1, 1, _TS), lambda b, st: (b, 0, st)),
            pl.BlockSpec((1, 1, _TS), lambda b, st: (b, 0, st)),
            pl.BlockSpec((1, _TS, 1), lambda b, st: (b, st, 0)),
            pl.BlockSpec((1, _TS, 1), lambda b, st: (b, st, 0)),
        ],
        out_specs=[
            pl.BlockSpec((1, _TS, _K), lambda b, st: (b, st, 0)),
            pl.BlockSpec((1, _TS, _K), lambda b, st: (b, st, 0)),
            pl.BlockSpec((1, _TS, D), lambda b, st: (b, st, 0)),
        ],
        out_shape=(
            jax.ShapeDtypeStruct((B, S, _K), jnp.int32),
            jax.ShapeDtypeStruct((B, S, _K), jnp.int32),
            jax.ShapeDtypeStruct((B, S, D), jnp.float32),
        ),
    )(wx0, wx1, pmat, xcol, ycol, sxr, syr, sxc, syc)


def _gt_body(wf_ref, wx0_ref, wx1_ref, x_ref, xyzt_ref, g_ref):
    xb = x_ref[0]
    g = lax.dot_general(
        xb, wf_ref[...], (((0,), (1,)), ((), ())),
        preferred_element_type=jnp.float32,
    )
    xsv = xyzt_ref[0, :, 0:1]
    ysv = xyzt_ref[0, :, 1:2]
    g = g + (xsv * wx0_ref[...] + ysv * wx1_ref[...]) * (1.0 / _RADIUS)
    g_ref[0] = jnp.concatenate(
        [g, jnp.zeros_like(g)], axis=1)


def _gt_call(wf, wx0, wx1, x, xyzt):
    B, D, N = x.shape
    return pl.pallas_call(
        _gt_body,
        grid=(B,),
        in_specs=[
            pl.BlockSpec((D, D), lambda b: (0, 0)),
            pl.BlockSpec((1, D), lambda b: (0, 0)),
            pl.BlockSpec((1, D), lambda b: (0, 0)),
            pl.BlockSpec((1, D, N), lambda b: (b, 0, 0)),
            pl.BlockSpec((1, N, 2), lambda b: (b, 0, 0)),
        ],
        out_specs=pl.BlockSpec((1, N, 2 * D), lambda b: (b, 0, 0)),
        out_shape=jax.ShapeDtypeStruct((B, N, 2 * D), jnp.float32),
    )(wf, wx0, wx1, x, xyzt)


def _sc_gather_max(table, fidx, ps, gamma, beta, B, S, D):
    SPW = (B * S) // _NW
    CH = 8
    NCH = SPW // CH
    ROWS = CH * _K
    mesh = plsc.VectorSubcoreMesh(core_axis_name="c", subcore_axis_name="s")

    @functools.partial(
        pl.kernel,
        out_type=jax.ShapeDtypeStruct((B * S, D), jnp.float32),
        mesh=mesh,
        scratch_types=[
            pltpu.VMEM((SPW * _K,), jnp.int32),
            pltpu.VMEM((ROWS, 2 * D), jnp.float32),
            pltpu.VMEM((ROWS, 2 * D), jnp.float32),
            pltpu.VMEM((SPW, D), jnp.float32),
            pltpu.VMEM((D,), jnp.float32),
            pltpu.VMEM((D,), jnp.float32),
            pltpu.VMEM((SPW, D), jnp.float32),
            pltpu.SemaphoreType.DMA,
            pltpu.SemaphoreType.DMA,
        ],
    )
    def k(table_h, fidx_h, ps_h, gamma_h, beta_h, out_h,
          idx_v, rows0, rows1, psb, gmv, btv, resb, sem0, sem1):
        cid = lax.axis_index("c")
        sid = lax.axis_index("s")
        wid = sid * 2 + cid
        base = wid * SPW
        pltpu.sync_copy(fidx_h.at[pl.ds(base * _K, SPW * _K)], idx_v)
        pltpu.sync_copy(ps_h.at[pl.ds(base, SPW)], psb)
        pltpu.sync_copy(gamma_h, gmv)
        pltpu.sync_copy(beta_h, btv)

        def start(c, buf, sem):
            pltpu.async_copy(
                table_h.at[idx_v.at[pl.ds(c * ROWS, ROWS)]], buf, sem)

        def wait(c, buf, sem):
            pltpu.make_async_copy(
                table_h.at[idx_v.at[pl.ds(c * ROWS, ROWS)]], buf, sem).wait()

        def compute(c, buf):
            for i in range(CH):
                s_local = c * CH + i
                for f in range(D // 16):
                    fo = f * 16
                    acc = buf[i * _K, pl.ds(fo, 16)]
                    for r in range(1, _K):
                        acc = jnp.maximum(acc, buf[i * _K + r, pl.ds(fo, 16)])
                    acc = acc - psb[s_local, pl.ds(fo, 16)]
                    acc = acc * (gmv[pl.ds(fo, 16)] * _BN_INV) + btv[pl.ds(fo, 16)]
                    acc = jnp.maximum(acc, 0.0)
                    resb[s_local, pl.ds(fo, 16)] = acc

        start(0, rows0, sem0)
        start(1, rows1, sem1)

        def loop_body(g, carry):
            c0 = g * 2
            wait(c0, rows0, sem0)
            compute(c0, rows0)

            @pl.when(g < NCH // 2 - 1)
            def _():
                start(c0 + 2, rows0, sem0)

            wait(c0 + 1, rows1, sem1)
            compute(c0 + 1, rows1)

            @pl.when(g < NCH // 2 - 1)
            def _():
                start(c0 + 3, rows1, sem1)

            return carry

        lax.fori_loop(0, NCH // 2, loop_body, 0)
        pltpu.sync_copy(resb, out_h.at[pl.ds(base, SPW)])

    return k(table, fidx, ps, gamma, beta)


def _tr_body(m_ref, out_ref):
    out_ref[0] = jnp.transpose(m_ref[0], (1, 0))


def _tr_call(m, B, S, D):
    return pl.pallas_call(
        _tr_body,
        grid=(B,),
        in_specs=[pl.BlockSpec((1, S, D), lambda b: (b, 0, 0))],
        out_specs=pl.BlockSpec((1, D, S), lambda b: (b, 0, 0)),
        out_shape=jax.ShapeDtypeStruct((B, D, S), jnp.float32),
    )(m)


def kernel(x, xyz, W, bn_gamma, bn_beta):
    B, D, N = x.shape
    S = N // _STRIDE
    xs = xyz[:, 0, :]
    ys = xyz[:, 1, :]
    sample_idx, sx, sy = _fps_call(xs, ys, S)
    sample_xyz = jnp.stack([sx, sy], axis=1)
    wf = W[:, :D]
    wx0 = W[:, D][None, :]
    wx1 = W[:, D + 1][None, :]
    n_ar = np.arange(N)
    pmat_np = np.zeros((N // _PB, N), np.float32)
    pmat_np[n_ar // _PB, n_ar] = np.float32(2.0) ** (n_ar % _PB)
    pmat = jnp.asarray(pmat_np)
    nidx, fidx, ps = _bq_call(wx0, wx1, pmat,
                              xs[:, :, None], ys[:, :, None],
                              sx[:, None, :], sy[:, None, :],
                              sx[:, :, None], sy[:, :, None])
    g = _gt_call(wf, wx0, wx1, x, jnp.transpose(xyz, (0, 2, 1)))
    m = _sc_gather_max(
        g.reshape(B * N, 2 * D),
        fidx.reshape(B * S * _K),
        ps.reshape(B * S, D),
        bn_gamma, bn_beta, B, S, D,
    )
    out = _tr_call(m.reshape(B, S, D), B, S, D)
    return out, sample_xyz, sample_idx, nidx


# speedup vs baseline: 32.3644x; 1.0009x over previous
import functools

import jax
import jax.numpy as jnp
import numpy as np
from jax import lax
from jax.experimental import pallas as pl
from jax.experimental.pallas import tpu as pltpu
from jax.experimental.pallas import tpu_sc as plsc

_RADIUS = 0.2
_R2 = _RADIUS ** 2
_K = 16
_STRIDE = 4
_EPS = 1e-5
_BN_INV = float(1.0 / np.sqrt(np.float32(1.0 + _EPS)))

_NW = 32
_TS = 128


_FW = 1024
_FB = 128


def _fps_body(xs_ref, ys_ref, idx_ref, sx_ref, sy_ref, dist_ref):
    B, N = xs_ref.shape
    S = idx_ref.shape[1]
    NC = N // _FW
    for c in range(NC):
        dist_ref[:, pl.ds(c * _FW, _FW)] = jnp.full((B, _FW), 1e10,
                                                    jnp.float32)
    lane_b = lax.broadcasted_iota(jnp.int32, (B, _FB), 1)

    def step(t, carry):
        far, cx, cy, bufi, bufx, bufy = carry
        tm = jnp.bitwise_and(t, _FB - 1)
        hit = lane_b == tm
        bufi = jnp.where(hit, far, bufi)
        bufx = jnp.where(hit, cx, bufx)
        bufy = jnp.where(hit, cy, bufy)

        ms, idxs, cxs, cys = [], [], [], []
        for c in range(NC):
            sl = pl.ds(c * _FW, _FW)
            xc = xs_ref[:, sl]
            yc = ys_ref[:, sl]
            dc = dist_ref[:, sl]
            dxc = xc - cx
            dyc = yc - cy
            dd = dxc * dxc + dyc * dyc
            dc = jnp.where(dd < dc, dd, dc)
            dist_ref[:, sl] = dc
            lane_c = lax.broadcasted_iota(jnp.int32, (B, _FW), 1) + c * _FW
            mc = jnp.max(dc, axis=1, keepdims=True)
            ic = jnp.min(jnp.where(dc == mc, lane_c, N), axis=1,
                         keepdims=True)
            oh = lane_c == ic
            ms.append(mc)
            idxs.append(ic)
            cxs.append(jnp.sum(jnp.where(oh, xc, 0.0), axis=1, keepdims=True))
            cys.append(jnp.sum(jnp.where(oh, yc, 0.0), axis=1, keepdims=True))
        gmax = ms[0]
        for c in range(1, NC):
            gmax = jnp.maximum(gmax, ms[c])
        gidx = None
        for c in range(NC):
            cand = jnp.where(ms[c] == gmax, idxs[c], N)
            gidx = cand if gidx is None else jnp.minimum(gidx, cand)
        ncx = jnp.zeros((B, 1), jnp.float32)
        ncy = jnp.zeros((B, 1), jnp.float32)
        for c in range(NC):
            selc = (ms[c] == gmax) & (idxs[c] == gidx)
            ncx = ncx + jnp.where(selc, cxs[c], 0.0)
            ncy = ncy + jnp.where(selc, cys[c], 0.0)

        blk = pl.multiple_of(jnp.bitwise_and(t, ~(_FB - 1)), _FB)
        idx_ref[:, pl.ds(blk, _FB)] = bufi
        sx_ref[:, pl.ds(blk, _FB)] = bufx
        sy_ref[:, pl.ds(blk, _FB)] = bufy
        return gidx, ncx, ncy, bufi, bufx, bufy

    first_m = lane_b == 0
    init = (
        jnp.min(jnp.where(first_m, lane_b, N), axis=1, keepdims=True),
        jnp.sum(jnp.where(first_m, xs_ref[:, 0:_FB], 0.0), axis=1,
                keepdims=True),
        jnp.sum(jnp.where(first_m, ys_ref[:, 0:_FB], 0.0), axis=1,
                keepdims=True),
        xs_ref[:, 0:_FB].astype(jnp.int32),
        xs_ref[:, 0:_FB],
        ys_ref[:, 0:_FB],
    )
    lax.fori_loop(0, S, step, init)


def _fps_call(xs, ys, S):
    B, N = xs.shape
    return pl.pallas_call(
        _fps_body,
        out_shape=(
            jax.ShapeDtypeStruct((B, S), jnp.int32),
            jax.ShapeDtypeStruct((B, S), jnp.float32),
            jax.ShapeDtypeStruct((B, S), jnp.float32),
        ),
        scratch_shapes=[pltpu.VMEM((B, N), jnp.float32)],
    )(xs, ys)


_PB = 16


def _bq_body(wx0_ref, wx1_ref, pmat_ref, xc_ref, yc_ref, sxr_ref, syr_ref,
             sxc_ref, syc_ref, nidx_ref, fidx_ref, ps_ref):
    b = pl.program_id(0)
    N = xc_ref.shape[1]
    NL = N // _PB
    xc = xc_ref[0]
    yc = yc_ref[0]
    sxr = sxr_ref[0]
    syr = syr_ref[0]
    dx = xc - sxr
    dy = yc - syr
    d = dx * dx + dy * dy
    mask = jnp.where(d <= _R2, 1.0, 0.0)
    pk = lax.dot_general(
        pmat_ref[...], mask, (((1,), (0,)), ((), ())),
        preferred_element_type=jnp.float32,
    )
    p = pk.astype(jnp.int32)
    subl = lax.broadcasted_iota(jnp.int32, p.shape, 0)
    big = jnp.int32(0x7FFFFFFF)
    comb = jnp.where(p > 0, (subl << 16) + p, big)
    cols = []
    first = None
    for j in range(_K):
        mc = jnp.min(comb, axis=0, keepdims=True)
        v = jnp.bitwise_and(mc, 0xFFFF)
        mnl = lax.shift_right_logical(mc, 16)
        lsb = jnp.bitwise_and(v, -v)
        bpos = lax.shift_right_logical(
            lax.bitcast_convert_type(lsb.astype(jnp.float32), jnp.int32), 23
        ) - 127
        idx = mnl * _PB + bpos
        if first is None:
            first = idx
            cols.append(idx)
        else:
            cols.append(jnp.where(mnl < NL, idx, first))
        if j < _K - 1:
            oneh = subl == mnl
            newc = jnp.where(v == lsb, big, mc - lsb)
            comb = jnp.where(oneh, newc, comb)
    nidx = jnp.transpose(jnp.concatenate(cols, axis=0), (1, 0))
    nidx_ref[0] = nidx
    fidx_ref[0] = nidx + b * N
    sxv = sxc_ref[0]
    syv = syc_ref[0]
    ps_ref[0] = (sxv * wx0_ref[...] + syv * wx1_ref[...]) * (1.0 / _RADIUS)


def _bq_call(wx0, wx1, pmat, xcol, ycol, sxr, syr, sxc, syc):
    B, N, _ = xcol.shape
    S = sxr.shape[2]
    D = wx0.shape[1]
    NL = N // _PB
    grid = (B, S // _TS)
    return pl.pallas_call(
        _bq_body,
        grid=grid,
        compiler_params=pltpu.CompilerParams(
            dimension_semantics=("parallel", "parallel")),
        in_specs=[
            pl.BlockSpec((1, D), lambda b, st: (0, 0)),
            pl.BlockSpec((1, D), lambda b, st: (0, 0)),
            pl.BlockSpec((NL, N), lambda b, st: (0, 0)),
            pl.BlockSpec((1, N, 1), lambda b, st: (b, 0, 0)),
            pl.BlockSpec((1, N, 1), lambda b, st: (b, 0, 0)),
            pl.BlockSpec((1, 1, _TS), lambda b, st: (b, 0, st)),
            pl.BlockSpec((1, 1, _TS), lambda b, st: (b, 0, st)),
            pl.BlockSpec((1, _TS, 1), lambda b, st: (b, st, 0)),
            pl.BlockSpec((1, _TS, 1), lambda b, st: (b, st, 0)),
        ],
        out_specs=[
            pl.BlockSpec((1, _TS, _K), lambda b, st: (b, st, 0)),
            pl.BlockSpec((1, _TS, _K), lambda b, st: (b, st, 0)),
            pl.BlockSpec((1, _TS, D), lambda b, st: (b, st, 0)),
        ],
        out_shape=(
            jax.ShapeDtypeStruct((B, S, _K), jnp.int32),
            jax.ShapeDtypeStruct((B, S, _K), jnp.int32),
            jax.ShapeDtypeStruct((B, S, D), jnp.float32),
        ),
    )(wx0, wx1, pmat, xcol, ycol, sxr, syr, sxc, syc)


def _gt_body(wf_ref, wx0_ref, wx1_ref, x_ref, xyzt_ref, g_ref):
    xb = x_ref[0]
    g = lax.dot_general(
        xb, wf_ref[...], (((0,), (1,)), ((), ())),
        preferred_element_type=jnp.float32,
    )
    xsv = xyzt_ref[0, :, 0:1]
    ysv = xyzt_ref[0, :, 1:2]
    g = g + (xsv * wx0_ref[...] + ysv * wx1_ref[...]) * (1.0 / _RADIUS)
    g_ref[0] = jnp.concatenate(
        [g, jnp.zeros_like(g)], axis=1)


def _gt_call(wf, wx0, wx1, x, xyzt):
    B, D, N = x.shape
    return pl.pallas_call(
        _gt_body,
        grid=(B,),
        in_specs=[
            pl.BlockSpec((D, D), lambda b: (0, 0)),
            pl.BlockSpec((1, D), lambda b: (0, 0)),
            pl.BlockSpec((1, D), lambda b: (0, 0)),
            pl.BlockSpec((1, D, N), lambda b: (b, 0, 0)),
            pl.BlockSpec((1, N, 2), lambda b: (b, 0, 0)),
        ],
        out_specs=pl.BlockSpec((1, N, 2 * D), lambda b: (b, 0, 0)),
        out_shape=jax.ShapeDtypeStruct((B, N, 2 * D), jnp.float32),
    )(wf, wx0, wx1, x, xyzt)


def _sc_gather_max(table, fidx, ps, gamma, beta, B, S, D):
    SPW = (B * S) // _NW
    CH = 8
    NCH = SPW // CH
    ROWS = CH * _K
    mesh = plsc.VectorSubcoreMesh(core_axis_name="c", subcore_axis_name="s")

    @functools.partial(
        pl.kernel,
        out_type=jax.ShapeDtypeStruct((B * S, D), jnp.float32),
        mesh=mesh,
        scratch_types=[
            pltpu.VMEM((SPW * _K,), jnp.int32),
            pltpu.VMEM((ROWS, 2 * D), jnp.float32),
            pltpu.VMEM((ROWS, 2 * D), jnp.float32),
            pltpu.VMEM((SPW, D), jnp.float32),
            pltpu.VMEM((D,), jnp.float32),
            pltpu.VMEM((D,), jnp.float32),
            pltpu.VMEM((SPW, D), jnp.float32),
            pltpu.SemaphoreType.DMA,
            pltpu.SemaphoreType.DMA,
        ],
    )
    def k(table_h, fidx_h, ps_h, gamma_h, beta_h, out_h,
          idx_v, rows0, rows1, psb, gmv, btv, resb, sem0, sem1):
        cid = lax.axis_index("c")
        sid = lax.axis_index("s")
        wid = sid * 2 + cid
        base = wid * SPW
        pltpu.sync_copy(fidx_h.at[pl.ds(base * _K, SPW * _K)], idx_v)
        pltpu.sync_copy(ps_h.at[pl.ds(base, SPW)], psb)
        pltpu.sync_copy(gamma_h, gmv)
        pltpu.sync_copy(beta_h, btv)

        def start(c, buf, sem):
            pltpu.async_copy(
                table_h.at[idx_v.at[pl.ds(c * ROWS, ROWS)]], buf, sem)

        def wait(c, buf, sem):
            pltpu.make_async_copy(
                table_h.at[idx_v.at[pl.ds(c * ROWS, ROWS)]], buf, sem).wait()

        def compute(c, buf):
            for i in range(CH):
                s_local = c * CH + i
                for f in range(D // 16):
                    fo = f * 16
                    acc = buf[i * _K, pl.ds(fo, 16)]
                    for r in range(1, _K):
                        acc = jnp.maximum(acc, buf[i * _K + r, pl.ds(fo, 16)])
                    acc = acc - psb[s_local, pl.ds(fo, 16)]
                    acc = acc * (gmv[pl.ds(fo, 16)] * _BN_INV) + btv[pl.ds(fo, 16)]
                    acc = jnp.maximum(acc, 0.0)
                    resb[s_local, pl.ds(fo, 16)] = acc

        start(0, rows0, sem0)
        start(1, rows1, sem1)

        def loop_body(g, carry):
            c0 = g * 2
            wait(c0, rows0, sem0)
            compute(c0, rows0)

            @pl.when(g < NCH // 2 - 1)
            def _():
                start(c0 + 2, rows0, sem0)

            wait(c0 + 1, rows1, sem1)
            compute(c0 + 1, rows1)

            @pl.when(g < NCH // 2 - 1)
            def _():
                start(c0 + 3, rows1, sem1)

            return carry

        lax.fori_loop(0, NCH // 2, loop_body, 0)
        pltpu.sync_copy(resb, out_h.at[pl.ds(base, SPW)])

    return k(table, fidx, ps, gamma, beta)


def _tr_body(m_ref, out_ref):
    out_ref[0] = jnp.transpose(m_ref[0], (1, 0))


def _tr_call(m, B, S, D):
    return pl.pallas_call(
        _tr_body,
        grid=(B,),
        in_specs=[pl.BlockSpec((1, S, D), lambda b: (b, 0, 0))],
        out_specs=pl.BlockSpec((1, D, S), lambda b: (b, 0, 0)),
        out_shape=jax.ShapeDtypeStruct((B, D, S), jnp.float32),
    )(m)


def kernel(x, xyz, W, bn_gamma, bn_beta):
    B, D, N = x.shape
    S = N // _STRIDE
    xs = xyz[:, 0, :]
    ys = xyz[:, 1, :]
    sample_idx, sx, sy = _fps_call(xs, ys, S)
    sample_xyz = jnp.stack([sx, sy], axis=1)
    wf = W[:, :D]
    wx0 = W[:, D][None, :]
    wx1 = W[:, D + 1][None, :]
    n_ar = np.arange(N)
    pmat_np = np.zeros((N // _PB, N), np.float32)
    pmat_np[n_ar // _PB, n_ar] = np.float32(2.0) ** (n_ar % _PB)
    pmat = jnp.asarray(pmat_np)
    nidx, fidx, ps = _bq_call(wx0, wx1, pmat,
                              xs[:, :, None], ys[:, :, None],
                              sx[:, None, :], sy[:, None, :],
                              sx[:, :, None], sy[:, :, None])
    g = _gt_call(wf, wx0, wx1, x, jnp.transpose(xyz, (0, 2, 1)))
    m = _sc_gather_max(
        g.reshape(B * N, 2 * D),
        fidx.reshape(B * S * _K),
        ps.reshape(B * S, D),
        bn_gamma, bn_beta, B, S, D,
    )
    out = _tr_call(m.reshape(B, S, D), B, S, D)
    return out, sample_xyz, sample_idx, nidx


# speedup vs baseline: 32.7823x; 1.0129x over previous
import functools

import jax
import jax.numpy as jnp
import numpy as np
from jax import lax
from jax.experimental import pallas as pl
from jax.experimental.pallas import tpu as pltpu
from jax.experimental.pallas import tpu_sc as plsc

_RADIUS = 0.2
_R2 = _RADIUS ** 2
_K = 16
_STRIDE = 4
_EPS = 1e-5
_BN_INV = float(1.0 / np.sqrt(np.float32(1.0 + _EPS)))

_NW = 32
_TS = 128


_FW = 512
_FB = 128


def _fps_body(xs_ref, ys_ref, idx_ref, sx_ref, sy_ref, dist_ref):
    B, N = xs_ref.shape
    S = idx_ref.shape[1]
    NC = N // _FW
    for c in range(NC):
        dist_ref[:, pl.ds(c * _FW, _FW)] = jnp.full((B, _FW), 1e10,
                                                    jnp.float32)
    lane_b = lax.broadcasted_iota(jnp.int32, (B, _FB), 1)

    def step(t, carry):
        far, cx, cy, bufi, bufx, bufy = carry
        tm = jnp.bitwise_and(t, _FB - 1)
        hit = lane_b == tm
        bufi = jnp.where(hit, far, bufi)
        bufx = jnp.where(hit, cx, bufx)
        bufy = jnp.where(hit, cy, bufy)

        ms, idxs, cxs, cys = [], [], [], []
        for c in range(NC):
            sl = pl.ds(c * _FW, _FW)
            xc = xs_ref[:, sl]
            yc = ys_ref[:, sl]
            dc = dist_ref[:, sl]
            dxc = xc - cx
            dyc = yc - cy
            dd = dxc * dxc + dyc * dyc
            dc = jnp.where(dd < dc, dd, dc)
            dist_ref[:, sl] = dc
            lane_c = lax.broadcasted_iota(jnp.int32, (B, _FW), 1) + c * _FW
            mc = jnp.max(dc, axis=1, keepdims=True)
            ic = jnp.min(jnp.where(dc == mc, lane_c, N), axis=1,
                         keepdims=True)
            oh = lane_c == ic
            ms.append(mc)
            idxs.append(ic)
            cxs.append(jnp.sum(jnp.where(oh, xc, 0.0), axis=1, keepdims=True))
            cys.append(jnp.sum(jnp.where(oh, yc, 0.0), axis=1, keepdims=True))
        gmax = ms[0]
        for c in range(1, NC):
            gmax = jnp.maximum(gmax, ms[c])
        gidx = None
        for c in range(NC):
            cand = jnp.where(ms[c] == gmax, idxs[c], N)
            gidx = cand if gidx is None else jnp.minimum(gidx, cand)
        ncx = jnp.zeros((B, 1), jnp.float32)
        ncy = jnp.zeros((B, 1), jnp.float32)
        for c in range(NC):
            selc = (ms[c] == gmax) & (idxs[c] == gidx)
            ncx = ncx + jnp.where(selc, cxs[c], 0.0)
            ncy = ncy + jnp.where(selc, cys[c], 0.0)

        blk = pl.multiple_of(jnp.bitwise_and(t, ~(_FB - 1)), _FB)
        idx_ref[:, pl.ds(blk, _FB)] = bufi
        sx_ref[:, pl.ds(blk, _FB)] = bufx
        sy_ref[:, pl.ds(blk, _FB)] = bufy
        return gidx, ncx, ncy, bufi, bufx, bufy

    first_m = lane_b == 0
    init = (
        jnp.min(jnp.where(first_m, lane_b, N), axis=1, keepdims=True),
        jnp.sum(jnp.where(first_m, xs_ref[:, 0:_FB], 0.0), axis=1,
                keepdims=True),
        jnp.sum(jnp.where(first_m, ys_ref[:, 0:_FB], 0.0), axis=1,
                keepdims=True),
        xs_ref[:, 0:_FB].astype(jnp.int32),
        xs_ref[:, 0:_FB],
        ys_ref[:, 0:_FB],
    )
    lax.fori_loop(0, S, step, init)


def _fps_call(xs, ys, S):
    B, N = xs.shape
    return pl.pallas_call(
        _fps_body,
        out_shape=(
            jax.ShapeDtypeStruct((B, S), jnp.int32),
            jax.ShapeDtypeStruct((B, S), jnp.float32),
            jax.ShapeDtypeStruct((B, S), jnp.float32),
        ),
        scratch_shapes=[pltpu.VMEM((B, N), jnp.float32)],
    )(xs, ys)


_PB = 16


def _bq_body(wx0_ref, wx1_ref, pmat_ref, xc_ref, yc_ref, sxr_ref, syr_ref,
             sxc_ref, syc_ref, nidx_ref, fidx_ref, ps_ref):
    b = pl.program_id(0)
    N = xc_ref.shape[1]
    NL = N // _PB
    xc = xc_ref[0]
    yc = yc_ref[0]
    sxr = sxr_ref[0]
    syr = syr_ref[0]
    dx = xc - sxr
    dy = yc - syr
    d = dx * dx + dy * dy
    mask = jnp.where(d <= _R2, 1.0, 0.0)
    pk = lax.dot_general(
        pmat_ref[...], mask, (((1,), (0,)), ((), ())),
        preferred_element_type=jnp.float32,
    )
    p = pk.astype(jnp.int32)
    subl = lax.broadcasted_iota(jnp.int32, p.shape, 0)
    big = jnp.int32(0x7FFFFFFF)
    comb = jnp.where(p > 0, (subl << 16) + p, big)
    cols = []
    first = None
    for j in range(_K):
        mc = jnp.min(comb, axis=0, keepdims=True)
        v = jnp.bitwise_and(mc, 0xFFFF)
        mnl = lax.shift_right_logical(mc, 16)
        lsb = jnp.bitwise_and(v, -v)
        bpos = lax.shift_right_logical(
            lax.bitcast_convert_type(lsb.astype(jnp.float32), jnp.int32), 23
        ) - 127
        idx = mnl * _PB + bpos
        if first is None:
            first = idx
            cols.append(idx)
        else:
            cols.append(jnp.where(mnl < NL, idx, first))
        if j < _K - 1:
            oneh = subl == mnl
            newc = jnp.where(v == lsb, big, mc - lsb)
            comb = jnp.where(oneh, newc, comb)
    nidx = jnp.transpose(jnp.concatenate(cols, axis=0), (1, 0))
    nidx_ref[0] = nidx
    fidx_ref[0] = nidx + b * N
    sxv = sxc_ref[0]
    syv = syc_ref[0]
    ps_ref[0] = (sxv * wx0_ref[...] + syv * wx1_ref[...]) * (1.0 / _RADIUS)


def _bq_call(wx0, wx1, pmat, xcol, ycol, sxr, syr, sxc, syc):
    B, N, _ = xcol.shape
    S = sxr.shape[2]
    D = wx0.shape[1]
    NL = N // _PB
    grid = (B, S // _TS)
    return pl.pallas_call(
        _bq_body,
        grid=grid,
        compiler_params=pltpu.CompilerParams(
            dimension_semantics=("parallel", "parallel")),
        in_specs=[
            pl.BlockSpec((1, D), lambda b, st: (0, 0)),
            pl.BlockSpec((1, D), lambda b, st: (0, 0)),
            pl.BlockSpec((NL, N), lambda b, st: (0, 0)),
            pl.BlockSpec((1, N, 1), lambda b, st: (b, 0, 0)),
            pl.BlockSpec((1, N, 1), lambda b, st: (b, 0, 0)),
            pl.BlockSpec((1, 1, _TS), lambda b, st: (b, 0, st)),
            pl.BlockSpec((1, 1, _TS), lambda b, st: (b, 0, st)),
            pl.BlockSpec((1, _TS, 1), lambda b, st: (b, st, 0)),
            pl.BlockSpec((1, _TS, 1), lambda b, st: (b, st, 0)),
        ],
        out_specs=[
            pl.BlockSpec((1, _TS, _K), lambda b, st: (b, st, 0)),
            pl.BlockSpec((1, _TS, _K), lambda b, st: (b, st, 0)),
            pl.BlockSpec((1, _TS, D), lambda b, st: (b, st, 0)),
        ],
        out_shape=(
            jax.ShapeDtypeStruct((B, S, _K), jnp.int32),
            jax.ShapeDtypeStruct((B, S, _K), jnp.int32),
            jax.ShapeDtypeStruct((B, S, D), jnp.float32),
        ),
    )(wx0, wx1, pmat, xcol, ycol, sxr, syr, sxc, syc)


def _gt_body(wf_ref, wx0_ref, wx1_ref, x_ref, xyzt_ref, g_ref):
    xb = x_ref[0]
    g = lax.dot_general(
        xb, wf_ref[...], (((0,), (1,)), ((), ())),
        preferred_element_type=jnp.float32,
    )
    xsv = xyzt_ref[0, :, 0:1]
    ysv = xyzt_ref[0, :, 1:2]
    g = g + (xsv * wx0_ref[...] + ysv * wx1_ref[...]) * (1.0 / _RADIUS)
    g_ref[0] = jnp.concatenate([g, jnp.zeros_like(g)], axis=1)


def _gt_call(wf, wx0, wx1, x, xyzt):
    B, D, N = x.shape
    return pl.pallas_call(
        _gt_body,
        grid=(B,),
        in_specs=[
            pl.BlockSpec((D, D), lambda b: (0, 0)),
            pl.BlockSpec((1, D), lambda b: (0, 0)),
            pl.BlockSpec((1, D), lambda b: (0, 0)),
            pl.BlockSpec((1, D, N), lambda b: (b, 0, 0)),
            pl.BlockSpec((1, N, 2), lambda b: (b, 0, 0)),
        ],
        out_specs=pl.BlockSpec((1, N, 2 * D), lambda b: (b, 0, 0)),
        out_shape=jax.ShapeDtypeStruct((B, N, 2 * D), jnp.float32),
    )(wf, wx0, wx1, x, xyzt)


def _sc_gather_max(table, fidx, ps, gamma, beta, B, S, D):
    SPW = (B * S) // _NW
    CH = 8
    NCH = SPW // CH
    ROWS = CH * _K
    mesh = plsc.VectorSubcoreMesh(core_axis_name="c", subcore_axis_name="s")

    @functools.partial(
        pl.kernel,
        out_type=jax.ShapeDtypeStruct((B * S, D), jnp.float32),
        mesh=mesh,
        scratch_types=[
            pltpu.VMEM((SPW * _K,), jnp.int32),
            pltpu.VMEM((ROWS, 2 * D), jnp.float32),
            pltpu.VMEM((ROWS, 2 * D), jnp.float32),
            pltpu.VMEM((SPW, D), jnp.float32),
            pltpu.VMEM((D,), jnp.float32),
            pltpu.VMEM((D,), jnp.float32),
            pltpu.VMEM((SPW, D), jnp.float32),
            pltpu.SemaphoreType.DMA,
            pltpu.SemaphoreType.DMA,
        ],
    )
    def k(table_h, fidx_h, ps_h, gamma_h, beta_h, out_h,
          idx_v, rows0, rows1, psb, gmv, btv, resb, sem0, sem1):
        cid = lax.axis_index("c")
        sid = lax.axis_index("s")
        wid = sid * 2 + cid
        base = wid * SPW
        pltpu.sync_copy(fidx_h.at[pl.ds(base * _K, SPW * _K)], idx_v)
        pltpu.sync_copy(ps_h.at[pl.ds(base, SPW)], psb)
        pltpu.sync_copy(gamma_h, gmv)
        pltpu.sync_copy(beta_h, btv)

        def start(c, buf, sem):
            pltpu.async_copy(
                table_h.at[idx_v.at[pl.ds(c * ROWS, ROWS)]], buf, sem)

        def wait(c, buf, sem):
            pltpu.make_async_copy(
                table_h.at[idx_v.at[pl.ds(c * ROWS, ROWS)]], buf, sem).wait()

        def compute(c, buf):
            for i in range(CH):
                s_local = c * CH + i
                for f in range(D // 16):
                    fo = f * 16
                    acc = buf[i * _K, pl.ds(fo, 16)]
                    for r in range(1, _K):
                        acc = jnp.maximum(acc, buf[i * _K + r, pl.ds(fo, 16)])
                    acc = acc - psb[s_local, pl.ds(fo, 16)]
                    acc = acc * (gmv[pl.ds(fo, 16)] * _BN_INV) + btv[pl.ds(fo, 16)]
                    acc = jnp.maximum(acc, 0.0)
                    resb[s_local, pl.ds(fo, 16)] = acc

        start(0, rows0, sem0)
        start(1, rows1, sem1)

        def loop_body(g, carry):
            c0 = g * 2
            wait(c0, rows0, sem0)
            compute(c0, rows0)

            @pl.when(g < NCH // 2 - 1)
            def _():
                start(c0 + 2, rows0, sem0)

            wait(c0 + 1, rows1, sem1)
            compute(c0 + 1, rows1)

            @pl.when(g < NCH // 2 - 1)
            def _():
                start(c0 + 3, rows1, sem1)

            return carry

        lax.fori_loop(0, NCH // 2, loop_body, 0)
        pltpu.sync_copy(resb, out_h.at[pl.ds(base, SPW)])

    return k(table, fidx, ps, gamma, beta)


def _tr_body(m_ref, out_ref):
    out_ref[0] = jnp.transpose(m_ref[0], (1, 0))


def _tr_call(m, B, S, D):
    return pl.pallas_call(
        _tr_body,
        grid=(B,),
        in_specs=[pl.BlockSpec((1, S, D), lambda b: (b, 0, 0))],
        out_specs=pl.BlockSpec((1, D, S), lambda b: (b, 0, 0)),
        out_shape=jax.ShapeDtypeStruct((B, D, S), jnp.float32),
    )(m)


def kernel(x, xyz, W, bn_gamma, bn_beta):
    B, D, N = x.shape
    S = N // _STRIDE
    xs = xyz[:, 0, :]
    ys = xyz[:, 1, :]
    sample_idx, sx, sy = _fps_call(xs, ys, S)
    sample_xyz = jnp.stack([sx, sy], axis=1)
    wf = W[:, :D]
    wx0 = W[:, D][None, :]
    wx1 = W[:, D + 1][None, :]
    n_ar = np.arange(N)
    pmat_np = np.zeros((N // _PB, N), np.float32)
    pmat_np[n_ar // _PB, n_ar] = np.float32(2.0) ** (n_ar % _PB)
    pmat = jnp.asarray(pmat_np)
    nidx, fidx, ps = _bq_call(wx0, wx1, pmat,
                              xs[:, :, None], ys[:, :, None],
                              sx[:, None, :], sy[:, None, :],
                              sx[:, :, None], sy[:, :, None])
    g = _gt_call(wf, wx0, wx1, x, jnp.transpose(xyz, (0, 2, 1)))
    m = _sc_gather_max(
        g.reshape(B * N, 2 * D),
        fidx.reshape(B * S * _K),
        ps.reshape(B * S, D),
        bn_gamma, bn_beta, B, S, D,
    )
    out = _tr_call(m.reshape(B, S, D), B, S, D)
    return out, sample_xyz, sample_idx, nidx
